# baseline (speedup 1.0000x reference)
import functools

import jax
import jax.numpy as jnp
from jax import lax
from jax.experimental import pallas as pl
from jax.experimental.pallas import tpu as pltpu
from jax.experimental.pallas import tpu_sc as plsc

B, L, A = 128, 512, 1024
AP = 1032
B_SC = 44
B_TC = B - B_SC
BB = 7
NW = 32
RPW = B_SC * L // NW
CH = 32
NBUF = 3
RU = 4
JU = 8
NLANE = 16


def _tc_matvec_kernel(d_ref, c_ref, o_ref):
    d = d_ref[...]
    w = c_ref[:, :, :A]
    bias = c_ref[:, :, A:A + 1]
    acc = jnp.sum(d * w, axis=-1)
    o_ref[...] = acc[:, None, :] + bias


def _tc_matvec(dict, coef):
    grid = (B_TC // BB,)
    out = pl.pallas_call(
        _tc_matvec_kernel,
        grid=grid,
        in_specs=[
            pl.BlockSpec((BB, L, A), lambda i: (i, 0, 0)),
            pl.BlockSpec((BB, 1, A + 1), lambda i: (i, 0, 0)),
        ],
        out_specs=pl.BlockSpec((BB, 1, L), lambda i: (i, 0, 0)),
        out_shape=jax.ShapeDtypeStruct((B_TC, 1, L), jnp.float32),
    )(dict, coef[:, None, :])
    return out.reshape(B_TC, L)


def _sc_matvec(dict, coef_pad):
    mesh = plsc.VectorSubcoreMesh(core_axis_name="c", subcore_axis_name="s")

    @functools.partial(
        pl.kernel,
        mesh=mesh,
        out_type=jax.ShapeDtypeStruct((B_SC * L,), jnp.float32),
        scratch_types=[
            pltpu.VMEM((AP,), jnp.float32),
            pltpu.VMEM((AP,), jnp.float32),
            pltpu.VMEM((AP,), jnp.float32),
            pltpu.VMEM((NBUF, CH, A), jnp.float32),
            pltpu.VMEM((RPW,), jnp.float32),
            pltpu.SemaphoreType.DMA,
            pltpu.SemaphoreType.DMA,
            pltpu.SemaphoreType.DMA,
            pltpu.SemaphoreType.DMA,
            pltpu.SemaphoreType.DMA,
            pltpu.SemaphoreType.DMA,
        ],
    )
    def sc_k(d_hbm, w_hbm, out_hbm, wv0, wv1, wv2, ring, out_v,
             d0, d1, d2, w0, w1, w2):
        c = lax.axis_index("c")
        s = lax.axis_index("s")
        wid = s * 2 + c

        n_chunks = RPW // CH
        dsems = (d0, d1, d2)
        wsems = (w0, w1, w2)
        wbufs = (wv0, wv1, wv2)
        lane_i = lax.iota(jnp.int32, NLANE)

        def batch_row(ci):
            flat = wid * RPW + ci * CH
            return B_TC + flat // L, flat % L

        def issue(ci):
            b, r = batch_row(ci)
            slot = ci % NBUF
            pltpu.async_copy(d_hbm.at[b, pl.ds(r, CH), :], ring.at[slot],
                             dsems[slot])
            pltpu.async_copy(w_hbm.at[b], wbufs[slot], wsems[slot])

        for ci in range(NBUF - 1):
            issue(ci)

        def run_chunk(ci):
            slot = ci % NBUF
            buf = ring.at[slot]
            w_v = wbufs[slot]
            bias_s = w_v[pl.ds(AP - NLANE, NLANE)][NLANE - (AP - A)]

            def row_group(g, carry):
                def subgroup(sg, out_vec):
                    r = g * NLANE + sg * RU

                    def j_block(jj, accs):
                        accs = list(accs)
                        for dj in range(JU):
                            off = (jj * JU + dj) * NLANE
                            wj = w_v[pl.ds(off, NLANE)]
                            for k in range(RU):
                                accs[k] = accs[k] + buf[r + k, pl.ds(off, NLANE)] * wj
                        return tuple(accs)

                    accs = lax.fori_loop(
                        0, A // (NLANE * JU), j_block,
                        tuple(jnp.zeros((NLANE,), jnp.float32) for _ in range(RU)),
                    )
                    for k in range(RU):
                        v = accs[k]
                        for sh in (8, 4, 2, 1):
                            v = v + v.at[lane_i ^ sh].get(mode="promise_in_bounds")
                        out_vec = jnp.where(lane_i == sg * RU + k, v + bias_s,
                                            out_vec)
                    return out_vec

                out_vec = lax.fori_loop(
                    0, NLANE // RU, subgroup, jnp.zeros((NLANE,), jnp.float32)
                )
                out_v[pl.ds(ci * CH + g * NLANE, NLANE)] = out_vec
                return carry

            lax.fori_loop(0, CH // NLANE, row_group, 0)

        for ci in range(n_chunks):
            b, r = batch_row(ci)
            slot = ci % NBUF
            pltpu.make_async_copy(d_hbm.at[b, pl.ds(r, CH), :], ring.at[slot],
                                  dsems[slot]).wait()
            pltpu.make_async_copy(w_hbm.at[b], wbufs[slot],
                                  wsems[slot]).wait()
            if ci + NBUF - 1 < n_chunks:
                issue(ci + NBUF - 1)
            run_chunk(ci)

        pltpu.sync_copy(out_v, out_hbm.at[pl.ds(wid * RPW, RPW)])

    return sc_k(dict, coef_pad)


def kernel(dict, coef):
    coef_pad = jnp.pad(coef, ((0, 0), (0, AP - (A + 1))))
    out_sc = _sc_matvec(dict, coef_pad).reshape(B_SC, L)
    out_tc = _tc_matvec(dict, coef)
    return jnp.concatenate([out_tc, out_sc], axis=0)[:, :, None]


# speedup vs baseline: 1.2993x; 1.2993x over previous
import jax
import jax.numpy as jnp
from jax.experimental import pallas as pl

B, L, A = 128, 512, 1024
BB = 8


def _matvec_bias_kernel(d_ref, c_ref, o_ref):
    d = d_ref[...]
    w = c_ref[:, :, :A]
    bias = c_ref[:, :, A:A + 1]
    acc = jnp.sum(d * w, axis=-1)
    o_ref[...] = acc[:, None, :] + bias


def kernel(dict, coef):
    grid = (B // BB,)
    out = pl.pallas_call(
        _matvec_bias_kernel,
        grid=grid,
        in_specs=[
            pl.BlockSpec((BB, L, A), lambda i: (i, 0, 0)),
            pl.BlockSpec((BB, 1, A + 1), lambda i: (i, 0, 0)),
        ],
        out_specs=pl.BlockSpec((BB, 1, L), lambda i: (i, 0, 0)),
        out_shape=jax.ShapeDtypeStruct((B, 1, L), jnp.float32),
    )(dict, coef[:, None, :])
    return out.reshape(B, L, 1)


# speedup vs baseline: 1.3183x; 1.0147x over previous
import jax
import jax.numpy as jnp
from jax.experimental import pallas as pl
from jax.experimental.pallas import tpu as pltpu

B, L, A = 128, 512, 1024
CB = 4
NB = 3


def _matvec_bias_kernel(d_hbm, c_ref, o_ref, ring, s0, s1, s2):
    sems = (s0, s1, s2)
    n_chunks = B // CB

    def issue(ci):
        slot = ci % NB
        pltpu.make_async_copy(
            d_hbm.at[pl.ds(ci * CB, CB)], ring.at[slot], sems[slot]
        ).start()

    for ci in range(NB - 1):
        issue(ci)

    for ci in range(n_chunks):
        slot = ci % NB
        pltpu.make_async_copy(
            d_hbm.at[pl.ds(ci * CB, CB)], ring.at[slot], sems[slot]
        ).wait()
        if ci + NB - 1 < n_chunks:
            issue(ci + NB - 1)
        d = ring[slot]
        c = c_ref[pl.ds(ci * CB, CB), :, :]
        acc = jnp.sum(d * c[:, :, :A], axis=-1)
        o_ref[pl.ds(ci * CB, CB), :, :] = acc[:, None, :] + c[:, :, A:A + 1]


def kernel(dict, coef):
    out = pl.pallas_call(
        _matvec_bias_kernel,
        in_specs=[
            pl.BlockSpec(memory_space=pl.ANY),
            pl.BlockSpec((B, 1, A + 1), lambda: (0, 0, 0)),
        ],
        out_specs=pl.BlockSpec((B, 1, L), lambda: (0, 0, 0)),
        out_shape=jax.ShapeDtypeStruct((B, 1, L), jnp.float32),
        scratch_shapes=[
            pltpu.VMEM((NB, CB, L, A), jnp.float32),
            pltpu.SemaphoreType.DMA,
            pltpu.SemaphoreType.DMA,
            pltpu.SemaphoreType.DMA,
        ],
    )(dict, coef[:, None, :])
    return out.reshape(B, L, 1)
